# baseline (speedup 1.0000x reference)
import functools

import jax
import jax.numpy as jnp
from jax import lax
from jax.experimental import pallas as pl
from jax.experimental.pallas import tpu as pltpu
from jax.experimental.pallas import tpu_sc as plsc


def _make_sc_gather(V2, D, B, S, NW, CR):
    mesh = plsc.VectorSubcoreMesh(core_axis_name="c", subcore_axis_name="s")
    info = plsc.get_sparse_core_info()
    NC = info.num_cores
    L = 16
    rows_per_w = B // NW
    n_chunks = rows_per_w // CR
    n_full = S // L
    tail = S - n_full * L
    tail_off = S - L
    n_vecs = n_full + (1 if tail else 0)

    @functools.partial(
        pl.kernel,
        mesh=mesh,
        compiler_params=pltpu.CompilerParams(use_tc_tiling_on_sc=False),
        out_type=jax.ShapeDtypeStruct((B, S, D), jnp.float32),
        scratch_types=[
            pltpu.VMEM((rows_per_w * S,), jnp.int32),
            pltpu.VMEM((2, CR, S, D), jnp.float32),
            pltpu.SemaphoreType.DMA((2,)),
            pltpu.SemaphoreType.DMA((2,)),
        ],
    )
    def gather(idx_hbm, table_hbm, out_hbm, idx_v, rows_v, gsem, ssem):
        wid = lax.axis_index("s") * NC + lax.axis_index("c")
        base = wid * rows_per_w
        pltpu.sync_copy(idx_hbm.at[wid], idx_v)

        def g_fire(b, j):
            def fire(r, carry):
                rg = (j * CR + r) * S
                for k in range(n_full):
                    vec = idx_v[pl.ds(rg + k * L, L)] * 2
                    pltpu.async_copy(
                        table_hbm.at[vec],
                        rows_v.at[b, r, pl.ds(k * L, L)],
                        gsem.at[b],
                    )
                if tail:
                    vec = idx_v[pl.ds(rg + tail_off, L)] * 2
                    pltpu.async_copy(
                        table_hbm.at[vec],
                        rows_v.at[b, r, pl.ds(tail_off, L)],
                        gsem.at[b],
                    )
                return carry

            lax.fori_loop(0, CR, fire, 0)

        def g_wait(b):
            for _ in range(n_vecs):
                pltpu.make_async_copy(
                    out_hbm.at[pl.ds(0, CR), pl.ds(0, L)],
                    rows_v.at[b, :, pl.ds(0, L)],
                    gsem.at[b],
                ).wait()

        def s_start(b, j):
            pltpu.async_copy(
                rows_v.at[b], out_hbm.at[pl.ds(base + j * CR, CR)], ssem.at[b]
            )

        def s_wait(b):
            pltpu.make_async_copy(
                rows_v.at[b], out_hbm.at[pl.ds(base, CR)], ssem.at[b]
            ).wait()

        g_fire(0, 0)
        g_fire(1, 1)
        for j in range(n_chunks):
            b = j & 1
            g_wait(b)
            s_start(b, j)
            if j + 2 < n_chunks:
                s_wait(b)
                g_fire(b, j + 2)
        s_wait(0)
        s_wait(1)

    return gather


def kernel(input, table):
    B, S = input.shape
    V, D = table.shape
    NW = 32
    CR = 16

    idx = input.reshape(NW, (B // NW) * S).astype(jnp.int32)
    table2 = jnp.pad(table, ((0, 0), (0, D))).reshape(2 * V, D)
    out = _make_sc_gather(2 * V, D, B, S, NW, CR)(idx, table2)
    return out
